# Initial kernel scaffold; baseline (speedup 1.0000x reference)
import jax
import jax.numpy as jnp
from jax.experimental import pallas as pl


def kernel(x, position_ids):
    raise NotImplementedError("write your pallas kernel here")


# speedup vs baseline: 5.1485x; 5.1485x over previous
import functools
import math

import jax
import jax.numpy as jnp
from jax.experimental import pallas as pl
from jax.experimental.pallas import tpu as pltpu

DIM = 128
HALF = DIM // 2
BASE = 10000.0
_NEG_LOG_BASE_OVER_HALF = -math.log(BASE) / HALF

ROWS_PER_BLOCK = 1024


def _rope_rows_kernel(pos_ref, cos_ref, sin_ref):
    rows = cos_ref.shape[0]
    pos = pos_ref[0]
    t = jnp.transpose(pos.astype(jnp.float32))
    lane = jax.lax.broadcasted_iota(jnp.int32, (1, DIM), 1)
    k = jnp.bitwise_and(lane, HALF - 1).astype(jnp.float32)
    inv_freq = jnp.exp(k * _NEG_LOG_BASE_OVER_HALF)
    angle = t * inv_freq
    cos_ref[...] = jnp.cos(angle)
    sin_ref[...] = jnp.sin(angle)


@functools.partial(jax.jit, static_argnames=("interpret",))
def _rope_tc(position_ids, interpret=False):
    b, s = position_ids.shape
    n = b * s
    rows = ROWS_PER_BLOCK
    nb = n // rows
    pos3 = position_ids.reshape(nb, 1, rows)
    out = pl.pallas_call(
        _rope_rows_kernel,
        grid=(nb,),
        in_specs=[pl.BlockSpec((1, 1, rows), lambda i: (i, 0, 0))],
        out_specs=[
            pl.BlockSpec((rows, DIM), lambda i: (i, 0)),
            pl.BlockSpec((rows, DIM), lambda i: (i, 0)),
        ],
        out_shape=[
            jax.ShapeDtypeStruct((n, DIM), jnp.float32),
            jax.ShapeDtypeStruct((n, DIM), jnp.float32),
        ],
        interpret=interpret,
    )(pos3)
    cos = out[0].reshape(b, 1, s, DIM)
    sin = out[1].reshape(b, 1, s, DIM)
    return cos, sin


def kernel(x, position_ids):
    del x
    return _rope_tc(position_ids)
